# baseline (speedup 1.0000x reference)
import functools

import jax
import jax.numpy as jnp
from jax import lax
from jax.experimental import pallas as pl
from jax.experimental.pallas import tpu as pltpu
from jax.experimental.pallas import tpu_sc as plsc

_MAGIC = 0x5CBA17E5


@functools.lru_cache(maxsize=None)
def _build(B, H, W):
    info = plsc.get_sparse_core_info()
    NC, NS, L = info.num_cores, info.num_subcores, info.num_lanes
    NW = NC * NS
    assert B % NW == 0
    bpw = B // NW
    C = 4096
    assert bpw % C == 0
    n_chunks = bpw // C
    SR = 4
    n_slabs = H // SR
    assert H % (SR * NW) == 0
    spw = n_slabs // NW

    mesh = plsc.VectorSubcoreMesh(core_axis_name="c", subcore_axis_name="s")

    scratch = {
        "fb0": pltpu.VMEM((SR, W), jnp.float32),
        "fb1": pltpu.VMEM((SR, W), jnp.float32),
        "sem_r0": pltpu.SemaphoreType.DMA,
        "sem_r1": pltpu.SemaphoreType.DMA,
        "sem_w0": pltpu.SemaphoreType.DMA,
        "sem_w1": pltpu.SemaphoreType.DMA,
        "idxall": pltpu.VMEM((bpw,), jnp.int32),
        "zx0": pltpu.VMEM((C,), jnp.float32),
        "zy0": pltpu.VMEM((C,), jnp.float32),
        "zx1": pltpu.VMEM((C,), jnp.float32),
        "zy1": pltpu.VMEM((C,), jnp.float32),
        "sem_z0": pltpu.SemaphoreType.DMA,
        "sem_z1": pltpu.SemaphoreType.DMA,
        "pv": pltpu.VMEM((4, L), jnp.float32),
        "sem_p": pltpu.SemaphoreType.DMA,
        "mb": pltpu.VMEM((L,), jnp.int32),
        "fl": pltpu.VMEM((NW, L), jnp.int32),
        "sem_f": pltpu.SemaphoreType.DMA,
        "done": pltpu.SMEM((1,), jnp.int32),
        "val0": pltpu.VMEM((C,), jnp.float32),
        "val1": pltpu.VMEM((C,), jnp.float32),
        "val2": pltpu.VMEM((C,), jnp.float32),
        "val3": pltpu.VMEM((C,), jnp.float32),
        "sem_g0": pltpu.SemaphoreType.DMA,
        "sem_g1": pltpu.SemaphoreType.DMA,
        "sem_g2": pltpu.SemaphoreType.DMA,
        "sem_g3": pltpu.SemaphoreType.DMA,
        "sem_o0": pltpu.SemaphoreType.DMA,
        "sem_o1": pltpu.SemaphoreType.DMA,
        "sem_o2": pltpu.SemaphoreType.DMA,
        "sem_o3": pltpu.SemaphoreType.DMA,
    }

    @functools.partial(
        pl.kernel,
        mesh=mesh,
        out_type=(
            jax.ShapeDtypeStruct((B,), jnp.float32),
            jax.ShapeDtypeStruct((H * W,), jnp.float32),
            jax.ShapeDtypeStruct((NW, L), jnp.int32),
        ),
        scratch_types=scratch,
        compiler_params=pltpu.CompilerParams(needs_layout_passes=False),
    )
    def k(zx_hbm, zy_hbm, d_hbm, p_hbm, out_hbm, tbl_hbm, flags_hbm, **s):
        wid = lax.axis_index("s") * NC + lax.axis_index("c")
        base = wid * bpw
        sbase = wid * spw
        pltpu.async_copy(p_hbm, s["pv"], s["sem_p"]).wait()
        shift_x = s["pv"][0]
        shift_y = s["pv"][1]
        scale_x = s["pv"][2]
        scale_y = s["pv"][3]
        szx = jnp.float32(H - 1)
        szy = jnp.float32(W - 1)

        fbufs = [(s["fb0"], s["sem_r0"], s["sem_w0"]),
                 (s["fb1"], s["sem_r1"], s["sem_w1"])]
        zbufs = [(s["zx0"], s["zy0"], s["sem_z0"]),
                 (s["zx1"], s["zy1"], s["sem_z1"])]
        vbufs = [(s[f"val{b}"], s[f"sem_g{b}"], s[f"sem_o{b}"]) for b in range(4)]

        def start_read(t):
            buf, sr, _ = fbufs[t % 2]
            sl = sbase + t
            pltpu.async_copy(d_hbm.at[pl.ds(sl * SR, SR), :], buf, sr)

        def wait_read(t):
            buf, sr, _ = fbufs[t % 2]
            sl = sbase + t
            pltpu.make_async_copy(d_hbm.at[pl.ds(sl * SR, SR), :], buf, sr).wait()

        def start_writes(t):
            buf, _, sw = fbufs[t % 2]
            sl = sbase + t
            for r in range(SR):
                pltpu.async_copy(buf.at[r], tbl_hbm.at[pl.ds((sl * SR + r) * W, W)], sw)

        def wait_writes(t):
            buf, _, sw = fbufs[t % 2]
            sl = sbase + t
            for r in range(SR):
                pltpu.make_async_copy(
                    buf.at[r], tbl_hbm.at[pl.ds((sl * SR + r) * W, W)], sw
                ).wait()

        def start_z(c):
            zx, zy, sem = zbufs[c % 2]
            cb = base + c * C
            pltpu.async_copy(zx_hbm.at[pl.ds(cb, C)], zx, sem)
            pltpu.async_copy(zy_hbm.at[pl.ds(cb, C)], zy, sem)

        def wait_z(c):
            zx, zy, sem = zbufs[c % 2]
            cb = base + c * C
            pltpu.make_async_copy(zx_hbm.at[pl.ds(cb, C)], zx, sem).wait()
            pltpu.make_async_copy(zy_hbm.at[pl.ds(cb, C)], zy, sem).wait()

        def compute_idx(c):
            zx, zy, _ = zbufs[c % 2]
            idxall = s["idxall"]

            def vec_body(j, carry):
                vx = zx[pl.ds(j * L, L)]
                vy = zy[pl.ds(j * L, L)]
                tx = jnp.clip((vx - shift_x) / scale_x, 0.0, 1.0)
                ty = jnp.clip((vy - shift_y) / scale_y, 0.0, 1.0)
                ix = (tx * szx).astype(jnp.int32)
                iy = (ty * szy).astype(jnp.int32)
                idxall[pl.ds(c * C + j * L, L)] = ix * W + iy
                return carry

            lax.fori_loop(0, C // L, vec_body, 0, unroll=4)

        start_z(0)
        start_z(1)
        start_read(0)
        zc = 0
        for t in range(spw):
            if t + 1 < spw:
                start_read(t + 1)
            if t >= 2:
                wait_writes(t - 2)
            wait_read(t)
            start_writes(t)
            if t % (spw // n_chunks) == 0 and zc < n_chunks:
                wait_z(zc)
                compute_idx(zc)
                if zc + 2 < n_chunks:
                    start_z(zc + 2)
                zc += 1
        while zc < n_chunks:
            wait_z(zc)
            compute_idx(zc)
            if zc + 2 < n_chunks:
                start_z(zc + 2)
            zc += 1
        for t in range(max(0, spw - 2), spw):
            wait_writes(t)

        s["mb"][...] = jnp.full((L,), _MAGIC, jnp.int32)
        pltpu.async_copy(s["mb"], flags_hbm.at[wid], s["sem_f"]).wait()
        s["done"][0] = jnp.int32(0)

        def poll_step(it, carry):
            @pl.when(s["done"][0] == 0)
            def _():
                pltpu.async_copy(flags_hbm, s["fl"], s["sem_f"]).wait()
                acc = s["fl"][0]
                for wv in range(1, NW):
                    acc = jnp.minimum(acc, s["fl"][wv])
                nok = jnp.max(
                    jnp.where(acc == jnp.int32(_MAGIC), jnp.int32(0), jnp.int32(1)),
                    axis=0,
                )
                s["done"][0] = jnp.where(nok > 0, jnp.int32(0), jnp.int32(1))

            return carry

        lax.fori_loop(0, 256, poll_step, 0)

        def start_gather(c):
            val, sem, _ = vbufs[c % 4]
            pltpu.async_copy(tbl_hbm.at[s["idxall"].at[pl.ds(c * C, C)]], val, sem)

        def wait_gather(c):
            val, sem, _ = vbufs[c % 4]
            pltpu.make_async_copy(
                tbl_hbm.at[s["idxall"].at[pl.ds(c * C, C)]], val, sem
            ).wait()

        def start_out(c):
            val, _, sem = vbufs[c % 4]
            cb = base + c * C
            pltpu.async_copy(val, out_hbm.at[pl.ds(cb, C)], sem)

        def wait_out(c):
            val, _, sem = vbufs[c % 4]
            cb = base + c * C
            pltpu.make_async_copy(val, out_hbm.at[pl.ds(cb, C)], sem).wait()

        start_gather(0)
        start_gather(1)
        for c in range(2, n_chunks):
            wait_gather(c - 2)
            start_out(c - 2)
            if c >= 4:
                wait_out(c - 4)
            start_gather(c)
        for c in range(max(0, n_chunks - 2), n_chunks):
            wait_gather(c)
            start_out(c)
        for c in range(max(0, n_chunks - 4), n_chunks):
            wait_out(c)

    return k


def kernel(z, density, scale, shift, image_size):
    B = z.shape[0]
    H, W = density.shape
    L = plsc.get_sparse_core_info().num_lanes
    zx = z[:, 0]
    zy = z[:, 1]
    params = jnp.concatenate(
        [
            jnp.broadcast_to(shift.reshape(2, 1), (2, L)),
            jnp.broadcast_to(scale.reshape(2, 1), (2, L)),
        ],
        axis=0,
    ).astype(jnp.float32)
    out, _, _ = _build(B, H, W)(zx, zy, density, params)
    return out


# speedup vs baseline: 1.1703x; 1.1703x over previous
import functools

import jax
import jax.numpy as jnp
from jax import lax
from jax.experimental import pallas as pl
from jax.experimental.pallas import tpu as pltpu
from jax.experimental.pallas import tpu_sc as plsc


@functools.lru_cache(maxsize=None)
def _build(B, H, W):
    info = plsc.get_sparse_core_info()
    NC, NS, L = info.num_cores, info.num_subcores, info.num_lanes
    NW = NC * NS
    assert B % NW == 0
    bpw = B // NW
    C = 4096
    assert bpw % C == 0
    n_chunks = bpw // C

    mesh = plsc.VectorSubcoreMesh(core_axis_name="c", subcore_axis_name="s")

    NB = 4
    scratch = {}
    for b in range(NB):
        scratch[f"zx{b}"] = pltpu.VMEM((C,), jnp.float32)
        scratch[f"zy{b}"] = pltpu.VMEM((C,), jnp.float32)
        scratch[f"idx{b}"] = pltpu.VMEM((C,), jnp.int32)
        scratch[f"val{b}"] = pltpu.VMEM((C,), jnp.float32)
        scratch[f"sem_z{b}"] = pltpu.SemaphoreType.DMA
        scratch[f"sem_g{b}"] = pltpu.SemaphoreType.DMA
        scratch[f"sem_o{b}"] = pltpu.SemaphoreType.DMA
    scratch["pv"] = pltpu.VMEM((4, L), jnp.float32)
    scratch["sem_p"] = pltpu.SemaphoreType.DMA

    @functools.partial(
        pl.kernel,
        mesh=mesh,
        out_type=jax.ShapeDtypeStruct((B,), jnp.float32),
        scratch_types=scratch,
    )
    def k(zx_hbm, zy_hbm, d_hbm, p_hbm, out_hbm, **s):
        wid = lax.axis_index("s") * NC + lax.axis_index("c")
        base = wid * bpw
        pltpu.async_copy(p_hbm, s["pv"], s["sem_p"]).wait()
        shift_x = s["pv"][0]
        shift_y = s["pv"][1]
        scale_x = s["pv"][2]
        scale_y = s["pv"][3]
        szx = jnp.float32(H - 1)
        szy = jnp.float32(W - 1)

        zbufs = [(s[f"zx{b}"], s[f"zy{b}"], s[f"sem_z{b}"]) for b in range(NB)]
        gbufs = [(s[f"idx{b}"], s[f"val{b}"], s[f"sem_g{b}"], s[f"sem_o{b}"])
                 for b in range(NB)]

        def start_z(c):
            zx, zy, sem = zbufs[c % NB]
            cb = base + c * C
            pltpu.async_copy(zx_hbm.at[pl.ds(cb, C)], zx, sem)
            pltpu.async_copy(zy_hbm.at[pl.ds(cb, C)], zy, sem)

        def wait_z(c):
            zx, zy, sem = zbufs[c % NB]
            cb = base + c * C
            pltpu.make_async_copy(zx_hbm.at[pl.ds(cb, C)], zx, sem).wait()
            pltpu.make_async_copy(zy_hbm.at[pl.ds(cb, C)], zy, sem).wait()

        def compute_idx(c):
            zx, zy, _ = zbufs[c % NB]
            idx = gbufs[c % NB][0]

            def vec_body(j, carry):
                vx = zx[pl.ds(j * L, L)]
                vy = zy[pl.ds(j * L, L)]
                tx = jnp.clip((vx - shift_x) / scale_x, 0.0, 1.0)
                ty = jnp.clip((vy - shift_y) / scale_y, 0.0, 1.0)
                ix = (tx * szx).astype(jnp.int32)
                iy = (ty * szy).astype(jnp.int32)
                idx[pl.ds(j * L, L)] = ix * W + iy
                return carry

            lax.fori_loop(0, C // L, vec_body, 0, unroll=4)

        def start_gather(c):
            idx, val, sem, _ = gbufs[c % NB]
            pltpu.async_copy(d_hbm.at[idx], val, sem)

        def wait_gather(c):
            idx, val, sem, _ = gbufs[c % NB]
            pltpu.make_async_copy(d_hbm.at[idx], val, sem).wait()

        def start_out(c):
            _, val, _, sem = gbufs[c % NB]
            cb = base + c * C
            pltpu.async_copy(val, out_hbm.at[pl.ds(cb, C)], sem)

        def wait_out(c):
            _, val, _, sem = gbufs[c % NB]
            cb = base + c * C
            pltpu.make_async_copy(val, out_hbm.at[pl.ds(cb, C)], sem).wait()

        start_z(0)
        start_z(1)
        wait_z(0)
        compute_idx(0)
        start_gather(0)
        wait_z(1)
        compute_idx(1)
        start_gather(1)
        for c in range(2, n_chunks):
            start_z(c)
            wait_z(c)
            compute_idx(c)
            wait_gather(c - 2)
            start_out(c - 2)
            if c >= 4:
                wait_out(c - 4)
            start_gather(c)
        for c in range(max(0, n_chunks - 2), n_chunks):
            wait_gather(c)
            start_out(c)
        for c in range(max(0, n_chunks - 4), n_chunks):
            wait_out(c)

    return k


def kernel(z, density, scale, shift, image_size):
    B = z.shape[0]
    H, W = density.shape
    L = plsc.get_sparse_core_info().num_lanes
    zx = z[:, 0]
    zy = z[:, 1]
    dflat = density.reshape(-1)
    params = jnp.concatenate(
        [
            jnp.broadcast_to(shift.reshape(2, 1), (2, L)),
            jnp.broadcast_to(scale.reshape(2, 1), (2, L)),
        ],
        axis=0,
    ).astype(jnp.float32)
    return _build(B, H, W)(zx, zy, dflat, params)
